# baseline (speedup 1.0000x reference)
import functools

import jax
import jax.numpy as jnp
from jax import lax
from jax.experimental import pallas as pl
from jax.experimental.pallas import tpu as pltpu
from jax.experimental.pallas import tpu_sc as plsc

Q = 1024
D = 16
K = 100000
KB = 2048
NB = -(-K // KB)

NC = 2
NS = 16
NW = NC * NS
BPW = Q // NW


def _argmin_body(q_ref, k_ref, idx_ref, bval, bidx, colf):
    j = pl.program_id(0)

    @pl.when(j == 0)
    def _():
        bval[...] = jnp.full((Q, 1), jnp.inf, jnp.float32)
        bidx[...] = jnp.zeros((Q, 1), jnp.int32)
        colf[...] = lax.broadcasted_iota(
            jnp.int32, (8, KB), 1).astype(jnp.float32)

    q = q_ref[...]
    kt = k_ref[...].T
    qkm2 = lax.dot_general(-2.0 * q, kt, (((1,), (0,)), ((), ())),
                           preferred_element_type=jnp.float32)
    q_sq = jnp.sum(q * q, axis=1, keepdims=True)
    k_sq = jnp.sum(kt * kt, axis=0, keepdims=True)
    s = (q_sq + k_sq) + qkm2

    def _reduce(s):
        bmin = jnp.min(s, axis=1, keepdims=True)
        s3 = s.reshape(Q // 8, 8, KB)
        bmin3 = bmin.reshape(Q // 8, 8, 1)
        col3 = colf[...].reshape(1, 8, KB)
        rel3 = jnp.min(jnp.where(s3 == bmin3, col3, float(KB)), axis=2,
                       keepdims=True)
        gidx = rel3.reshape(Q, 1).astype(jnp.int32) + j * KB
        better = bmin < bval[...]
        bval[...] = jnp.where(better, bmin, bval[...])
        bidx[...] = jnp.where(better, gidx, bidx[...])
        idx_ref[...] = bidx[...]

    @pl.when(j < NB - 1)
    def _():
        _reduce(s)

    @pl.when(j == NB - 1)
    def _():
        ncols = jnp.float32(K - (NB - 1) * KB)
        _reduce(jnp.where(colf[...].reshape(1, 8, KB) < ncols,
                          s.reshape(Q // 8, 8, KB),
                          jnp.inf).reshape(Q, KB))


def _tc_argmin(queries, keys):
    return pl.pallas_call(
        _argmin_body,
        grid=(NB,),
        in_specs=[
            pl.BlockSpec((Q, D), lambda j: (0, 0)),
            pl.BlockSpec((KB, D), lambda j: (j, 0)),
        ],
        out_specs=pl.BlockSpec((Q, 1), lambda j: (0, 0)),
        out_shape=jax.ShapeDtypeStruct((Q, 1), jnp.int32),
        scratch_shapes=[
            pltpu.VMEM((Q, 1), jnp.float32),
            pltpu.VMEM((Q, 1), jnp.int32),
            pltpu.VMEM((8, KB), jnp.float32),
        ],
    )(queries, keys)


def _sc_body(idx_hbm, q_hbm, gap_hbm, keys_hbm, out_hbm,
             idx_v, rows_v, q_v, g_v, o_v, sem):
    c = lax.axis_index("c")
    s = lax.axis_index("s")
    wid = s * NC + c
    base = wid * BPW
    pltpu.sync_copy(idx_hbm.at[pl.ds(base, BPW)], idx_v)
    cp = pltpu.async_copy(keys_hbm.at[idx_v], rows_v, sem)
    pltpu.sync_copy(q_hbm.at[pl.ds(base, BPW)], q_v)
    pltpu.sync_copy(gap_hbm.at[pl.ds(base, BPW)], g_v)
    cp.wait()
    for blk in range(BPW // 16):
        gv = g_v[pl.ds(blk * 16, 16)]
        for lane in range(16):
            i = blk * 16 + lane
            g = gv[lane]
            qrow = q_v[i]
            rrow = rows_v[i]
            o_v[i] = qrow + g * (rrow - qrow)
    pltpu.sync_copy(o_v, out_hbm.at[pl.ds(base, BPW)])


def _sc_interp(nn_idx, queries, gap, keys):
    mesh = plsc.VectorSubcoreMesh(
        core_axis_name="c", subcore_axis_name="s",
        num_cores=NC, num_subcores=NS)
    f = pl.kernel(
        _sc_body,
        out_type=jax.ShapeDtypeStruct((Q, D), jnp.float32),
        mesh=mesh,
        scratch_types=[
            pltpu.VMEM((BPW,), jnp.int32),
            pltpu.VMEM((BPW, D), jnp.float32),
            pltpu.VMEM((BPW, D), jnp.float32),
            pltpu.VMEM((BPW,), jnp.float32),
            pltpu.VMEM((BPW, D), jnp.float32),
            pltpu.SemaphoreType.DMA,
        ],
        compiler_params=pltpu.CompilerParams(use_tc_tiling_on_sc=False),
    )
    return f(nn_idx, queries, gap, keys)


def kernel(queries, keys, gap):
    nn_idx = _tc_argmin(queries, keys).reshape(Q)
    return _sc_interp(nn_idx, queries, gap, keys)


# speedup vs baseline: 1.1870x; 1.1870x over previous
import functools

import jax
import jax.numpy as jnp
from jax import lax
from jax.experimental import pallas as pl
from jax.experimental.pallas import tpu as pltpu
from jax.experimental.pallas import tpu_sc as plsc

Q = 1024
D = 16
K = 100000
KB = 2048
NB = -(-K // KB)

NC = 2
NS = 16
NW = NC * NS
BPW = Q // NW


def _argmin_body(q_ref, k_ref, idx_ref, bval, bidx, colf):
    j = pl.program_id(0)

    @pl.when(j == 0)
    def _():
        bval[...] = jnp.full((Q, 1), jnp.inf, jnp.float32)
        bidx[...] = jnp.zeros((Q, 1), jnp.int32)
        colf[...] = lax.broadcasted_iota(
            jnp.int32, (8, KB), 1).astype(jnp.float32)

    q = q_ref[...]
    kt = k_ref[...]
    qkm2 = lax.dot_general(-2.0 * q, kt, (((1,), (0,)), ((), ())),
                           preferred_element_type=jnp.float32)
    q_sq = jnp.sum(q * q, axis=1, keepdims=True)
    k_sq = jnp.sum(kt * kt, axis=0, keepdims=True)
    s = (q_sq + k_sq) + qkm2

    def _reduce(s):
        bmin = jnp.min(s, axis=1, keepdims=True)
        s3 = s.reshape(Q // 8, 8, KB)
        bmin3 = bmin.reshape(Q // 8, 8, 1)
        col3 = colf[...].reshape(1, 8, KB)
        rel3 = jnp.min(jnp.where(s3 == bmin3, col3, float(KB)), axis=2,
                       keepdims=True)
        gidx = rel3.reshape(Q, 1).astype(jnp.int32) + j * KB
        better = bmin < bval[...]
        bval[...] = jnp.where(better, bmin, bval[...])
        bidx[...] = jnp.where(better, gidx, bidx[...])
        idx_ref[...] = bidx[...]

    @pl.when(j < NB - 1)
    def _():
        _reduce(s)

    @pl.when(j == NB - 1)
    def _():
        ncols = jnp.float32(K - (NB - 1) * KB)
        _reduce(jnp.where(colf[...].reshape(1, 8, KB) < ncols,
                          s.reshape(Q // 8, 8, KB),
                          jnp.inf).reshape(Q, KB))


def _tc_argmin(queries, keys):
    return pl.pallas_call(
        _argmin_body,
        grid=(NB,),
        in_specs=[
            pl.BlockSpec((Q, D), lambda j: (0, 0)),
            pl.BlockSpec((D, KB), lambda j: (0, j)),
        ],
        out_specs=pl.BlockSpec((Q, 1), lambda j: (0, 0)),
        out_shape=jax.ShapeDtypeStruct((Q, 1), jnp.int32),
        scratch_shapes=[
            pltpu.VMEM((Q, 1), jnp.float32),
            pltpu.VMEM((Q, 1), jnp.int32),
            pltpu.VMEM((8, KB), jnp.float32),
        ],
    )(queries, keys)


def _sc_body(idx_hbm, q_hbm, gap_hbm, keys_hbm, out_hbm,
             idx_v, rows_v, q_v, g_v, o_v, sem):
    c = lax.axis_index("c")
    s = lax.axis_index("s")
    wid = s * NC + c
    base = wid * BPW
    pltpu.sync_copy(idx_hbm.at[pl.ds(base, BPW)], idx_v)
    cp = pltpu.async_copy(keys_hbm.at[idx_v], rows_v, sem)
    pltpu.sync_copy(q_hbm.at[pl.ds(base, BPW)], q_v)
    pltpu.sync_copy(gap_hbm.at[pl.ds(base, BPW)], g_v)
    cp.wait()
    for blk in range(BPW // 16):
        gv = g_v[pl.ds(blk * 16, 16)]
        for lane in range(16):
            i = blk * 16 + lane
            g = gv[lane]
            qrow = q_v[i]
            rrow = rows_v[i]
            o_v[i] = qrow + g * (rrow - qrow)
    pltpu.sync_copy(o_v, out_hbm.at[pl.ds(base, BPW)])


def _sc_interp(nn_idx, queries, gap, keys):
    mesh = plsc.VectorSubcoreMesh(
        core_axis_name="c", subcore_axis_name="s",
        num_cores=NC, num_subcores=NS)
    f = pl.kernel(
        _sc_body,
        out_type=jax.ShapeDtypeStruct((Q, D), jnp.float32),
        mesh=mesh,
        scratch_types=[
            pltpu.VMEM((BPW,), jnp.int32),
            pltpu.VMEM((BPW, D), jnp.float32),
            pltpu.VMEM((BPW, D), jnp.float32),
            pltpu.VMEM((BPW,), jnp.float32),
            pltpu.VMEM((BPW, D), jnp.float32),
            pltpu.SemaphoreType.DMA,
        ],
        compiler_params=pltpu.CompilerParams(use_tc_tiling_on_sc=False),
    )
    return f(nn_idx, queries, gap, keys)


def kernel(queries, keys, gap):
    nn_idx = _tc_argmin(queries, keys.T).reshape(Q)
    return _sc_interp(nn_idx, queries, gap, keys)


# speedup vs baseline: 1.2259x; 1.0327x over previous
import functools

import jax
import jax.numpy as jnp
from jax import lax
from jax.experimental import pallas as pl
from jax.experimental.pallas import tpu as pltpu
from jax.experimental.pallas import tpu_sc as plsc

Q = 1024
D = 16
K = 100000
KB = 4096
NB = -(-K // KB)

NC = 2
NS = 16
NW = NC * NS
BPW = Q // NW


def _argmin_body(q_ref, k_ref, idx_ref, bval, bidx, colf):
    j = pl.program_id(0)

    @pl.when(j == 0)
    def _():
        bval[...] = jnp.full((Q, 1), jnp.inf, jnp.float32)
        bidx[...] = jnp.zeros((Q, 1), jnp.int32)
        colf[...] = lax.broadcasted_iota(
            jnp.int32, (8, KB), 1).astype(jnp.float32)

    q = q_ref[...]
    kt = k_ref[...]
    qkm2 = lax.dot_general(-2.0 * q, kt, (((1,), (0,)), ((), ())),
                           preferred_element_type=jnp.float32)
    q_sq = jnp.sum(q * q, axis=1, keepdims=True)
    k_sq = jnp.sum(kt * kt, axis=0, keepdims=True)
    s = (q_sq + k_sq) + qkm2

    def _reduce(s):
        bmin = jnp.min(s, axis=1, keepdims=True)
        s3 = s.reshape(Q // 8, 8, KB)
        bmin3 = bmin.reshape(Q // 8, 8, 1)
        col3 = colf[...].reshape(1, 8, KB)
        rel3 = jnp.min(jnp.where(s3 == bmin3, col3, float(KB)), axis=2,
                       keepdims=True)
        gidx = rel3.reshape(Q, 1).astype(jnp.int32) + j * KB
        better = bmin < bval[...]
        bval[...] = jnp.where(better, bmin, bval[...])
        bidx[...] = jnp.where(better, gidx, bidx[...])
        idx_ref[...] = bidx[...]

    @pl.when(j < NB - 1)
    def _():
        _reduce(s)

    @pl.when(j == NB - 1)
    def _():
        ncols = jnp.float32(K - (NB - 1) * KB)
        _reduce(jnp.where(colf[...].reshape(1, 8, KB) < ncols,
                          s.reshape(Q // 8, 8, KB),
                          jnp.inf).reshape(Q, KB))


def _tc_argmin(queries, keys):
    return pl.pallas_call(
        _argmin_body,
        grid=(NB,),
        in_specs=[
            pl.BlockSpec((Q, D), lambda j: (0, 0)),
            pl.BlockSpec((D, KB), lambda j: (0, j)),
        ],
        out_specs=pl.BlockSpec((Q, 1), lambda j: (0, 0)),
        out_shape=jax.ShapeDtypeStruct((Q, 1), jnp.int32),
        scratch_shapes=[
            pltpu.VMEM((Q, 1), jnp.float32),
            pltpu.VMEM((Q, 1), jnp.int32),
            pltpu.VMEM((8, KB), jnp.float32),
        ],
    )(queries, keys)


def _sc_body(idx_hbm, q_hbm, gap_hbm, keys_hbm, out_hbm,
             idx_v, rows_v, q_v, g_v, o_v, sem):
    c = lax.axis_index("c")
    s = lax.axis_index("s")
    wid = s * NC + c
    base = wid * BPW
    pltpu.sync_copy(idx_hbm.at[pl.ds(base, BPW)], idx_v)
    cp = pltpu.async_copy(keys_hbm.at[idx_v], rows_v, sem)
    pltpu.sync_copy(q_hbm.at[pl.ds(base, BPW)], q_v)
    pltpu.sync_copy(gap_hbm.at[pl.ds(base, BPW)], g_v)
    cp.wait()
    for blk in range(BPW // 16):
        gv = g_v[pl.ds(blk * 16, 16)]
        for lane in range(16):
            i = blk * 16 + lane
            g = gv[lane]
            qrow = q_v[i]
            rrow = rows_v[i]
            o_v[i] = qrow + g * (rrow - qrow)
    pltpu.sync_copy(o_v, out_hbm.at[pl.ds(base, BPW)])


def _sc_interp(nn_idx, queries, gap, keys):
    mesh = plsc.VectorSubcoreMesh(
        core_axis_name="c", subcore_axis_name="s",
        num_cores=NC, num_subcores=NS)
    f = pl.kernel(
        _sc_body,
        out_type=jax.ShapeDtypeStruct((Q, D), jnp.float32),
        mesh=mesh,
        scratch_types=[
            pltpu.VMEM((BPW,), jnp.int32),
            pltpu.VMEM((BPW, D), jnp.float32),
            pltpu.VMEM((BPW, D), jnp.float32),
            pltpu.VMEM((BPW,), jnp.float32),
            pltpu.VMEM((BPW, D), jnp.float32),
            pltpu.SemaphoreType.DMA,
        ],
        compiler_params=pltpu.CompilerParams(use_tc_tiling_on_sc=False),
    )
    return f(nn_idx, queries, gap, keys)


def kernel(queries, keys, gap):
    nn_idx = _tc_argmin(queries, keys.T).reshape(Q)
    return _sc_interp(nn_idx, queries, gap, keys)


# speedup vs baseline: 1.2320x; 1.0050x over previous
import functools

import jax
import jax.numpy as jnp
from jax import lax
from jax.experimental import pallas as pl
from jax.experimental.pallas import tpu as pltpu
from jax.experimental.pallas import tpu_sc as plsc

Q = 1024
D = 16
K = 100000
KB = 6144
NB = -(-K // KB)

NC = 2
NS = 16
NW = NC * NS
BPW = Q // NW


def _argmin_body(q_ref, k_ref, idx_ref, bval, bidx, colf):
    j = pl.program_id(0)

    @pl.when(j == 0)
    def _():
        bval[...] = jnp.full((Q, 1), jnp.inf, jnp.float32)
        bidx[...] = jnp.zeros((Q, 1), jnp.int32)
        colf[...] = lax.broadcasted_iota(
            jnp.int32, (8, KB), 1).astype(jnp.float32)

    q = q_ref[...]
    kt = k_ref[...]
    qkm2 = lax.dot_general(-2.0 * q, kt, (((1,), (0,)), ((), ())),
                           preferred_element_type=jnp.float32)
    q_sq = jnp.sum(q * q, axis=1, keepdims=True)
    k_sq = jnp.sum(kt * kt, axis=0, keepdims=True)
    s = (q_sq + k_sq) + qkm2

    def _reduce(s):
        bmin = jnp.min(s, axis=1, keepdims=True)
        s3 = s.reshape(Q // 8, 8, KB)
        bmin3 = bmin.reshape(Q // 8, 8, 1)
        col3 = colf[...].reshape(1, 8, KB)
        rel3 = jnp.min(jnp.where(s3 == bmin3, col3, float(KB)), axis=2,
                       keepdims=True)
        gidx = rel3.reshape(Q, 1).astype(jnp.int32) + j * KB
        better = bmin < bval[...]
        bval[...] = jnp.where(better, bmin, bval[...])
        bidx[...] = jnp.where(better, gidx, bidx[...])
        idx_ref[...] = bidx[...]

    @pl.when(j < NB - 1)
    def _():
        _reduce(s)

    @pl.when(j == NB - 1)
    def _():
        ncols = jnp.float32(K - (NB - 1) * KB)
        _reduce(jnp.where(colf[...].reshape(1, 8, KB) < ncols,
                          s.reshape(Q // 8, 8, KB),
                          jnp.inf).reshape(Q, KB))


def _tc_argmin(queries, keys):
    return pl.pallas_call(
        _argmin_body,
        grid=(NB,),
        in_specs=[
            pl.BlockSpec((Q, D), lambda j: (0, 0)),
            pl.BlockSpec((D, KB), lambda j: (0, j)),
        ],
        out_specs=pl.BlockSpec((Q, 1), lambda j: (0, 0)),
        out_shape=jax.ShapeDtypeStruct((Q, 1), jnp.int32),
        scratch_shapes=[
            pltpu.VMEM((Q, 1), jnp.float32),
            pltpu.VMEM((Q, 1), jnp.int32),
            pltpu.VMEM((8, KB), jnp.float32),
        ],
    )(queries, keys)


def _sc_body(idx_hbm, q_hbm, gap_hbm, keys_hbm, out_hbm,
             idx_v, rows_v, q_v, g_v, o_v, sem):
    c = lax.axis_index("c")
    s = lax.axis_index("s")
    wid = s * NC + c
    base = wid * BPW
    pltpu.sync_copy(idx_hbm.at[pl.ds(base, BPW)], idx_v)
    cp = pltpu.async_copy(keys_hbm.at[idx_v], rows_v, sem)
    pltpu.sync_copy(q_hbm.at[pl.ds(base, BPW)], q_v)
    pltpu.sync_copy(gap_hbm.at[pl.ds(base, BPW)], g_v)
    cp.wait()
    for blk in range(BPW // 16):
        gv = g_v[pl.ds(blk * 16, 16)]
        for lane in range(16):
            i = blk * 16 + lane
            g = gv[lane]
            qrow = q_v[i]
            rrow = rows_v[i]
            o_v[i] = qrow + g * (rrow - qrow)
    pltpu.sync_copy(o_v, out_hbm.at[pl.ds(base, BPW)])


def _sc_interp(nn_idx, queries, gap, keys):
    mesh = plsc.VectorSubcoreMesh(
        core_axis_name="c", subcore_axis_name="s",
        num_cores=NC, num_subcores=NS)
    f = pl.kernel(
        _sc_body,
        out_type=jax.ShapeDtypeStruct((Q, D), jnp.float32),
        mesh=mesh,
        scratch_types=[
            pltpu.VMEM((BPW,), jnp.int32),
            pltpu.VMEM((BPW, D), jnp.float32),
            pltpu.VMEM((BPW, D), jnp.float32),
            pltpu.VMEM((BPW,), jnp.float32),
            pltpu.VMEM((BPW, D), jnp.float32),
            pltpu.SemaphoreType.DMA,
        ],
        compiler_params=pltpu.CompilerParams(use_tc_tiling_on_sc=False),
    )
    return f(nn_idx, queries, gap, keys)


def kernel(queries, keys, gap):
    nn_idx = _tc_argmin(queries, keys.T).reshape(Q)
    return _sc_interp(nn_idx, queries, gap, keys)
